# Initial kernel scaffold; baseline (speedup 1.0000x reference)
import jax
import jax.numpy as jnp
from jax.experimental import pallas as pl


def kernel(x, wm, bm, wt, bt, wa, ba, wsh, bsh):
    raise NotImplementedError("write your pallas kernel here")


# speedup vs baseline: 1.1050x; 1.1050x over previous
import jax
import jax.numpy as jnp
from jax.experimental import pallas as pl
from jax.experimental.pallas import tpu as pltpu


def _fused_kernel(x_ref, wm_ref, bm_ref, wg_ref, bg_ref,
                  ft_ref, va_ref, fsh_ref):
    x0 = x_ref[0]
    C = x0.shape[0]
    s = jnp.sum(x0, axis=1, keepdims=True)
    v = jnp.dot(wm_ref[...], s, preferred_element_type=jnp.float32)
    v = jnp.maximum(v + bm_ref[...], 0.0)
    g = jax.nn.sigmoid(
        jnp.dot(wg_ref[...], v, preferred_element_type=jnp.float32)
        + bg_ref[...])
    ft_ref[0] = g[0:C] * x0
    va_ref[0] = g[C:2 * C]
    fsh_ref[0] = g[2 * C:3 * C] * x0


def kernel(x, wm, bm, wt, bt, wa, ba, wsh, bsh):
    B, C, H, W = x.shape
    HW = H * W
    rC = wm.shape[1]

    wm_t = jnp.transpose(wm).astype(jnp.float32) / float(HW)
    bm_t = jnp.transpose(bm).astype(jnp.float32)
    wg_t = jnp.concatenate(
        [jnp.transpose(wt), jnp.transpose(wa), jnp.transpose(wsh)],
        axis=0).astype(jnp.float32)
    bg_t = jnp.concatenate(
        [jnp.transpose(bt), jnp.transpose(ba), jnp.transpose(bsh)],
        axis=0).astype(jnp.float32)

    x_flat = x.reshape(B, C, HW)

    ft, va, fsh = pl.pallas_call(
        _fused_kernel,
        out_shape=(
            jax.ShapeDtypeStruct((B, C, HW), x.dtype),
            jax.ShapeDtypeStruct((B, C, 1), jnp.float32),
            jax.ShapeDtypeStruct((B, C, HW), x.dtype),
        ),
        grid=(B,),
        in_specs=[
            pl.BlockSpec((1, C, HW), lambda b: (b, 0, 0)),
            pl.BlockSpec((rC, C), lambda b: (0, 0)),
            pl.BlockSpec((rC, 1), lambda b: (0, 0)),
            pl.BlockSpec((3 * C, rC), lambda b: (0, 0)),
            pl.BlockSpec((3 * C, 1), lambda b: (0, 0)),
        ],
        out_specs=(
            pl.BlockSpec((1, C, HW), lambda b: (b, 0, 0)),
            pl.BlockSpec((1, C, 1), lambda b: (b, 0, 0)),
            pl.BlockSpec((1, C, HW), lambda b: (b, 0, 0)),
        ),
        compiler_params=pltpu.CompilerParams(
            dimension_semantics=("parallel",),
            vmem_limit_bytes=48 * 1024 * 1024),
    )(x_flat, wm_t, bm_t, wg_t, bg_t)

    return (ft.reshape(B, C, H, W), va.reshape(B, C),
            fsh.reshape(B, C, H, W))
